# baseline (speedup 1.0000x reference)
import functools

import jax
import jax.numpy as jnp
from jax import lax
from jax.experimental import pallas as pl
from jax.experimental.pallas import tpu as pltpu
from jax.experimental.pallas import tpu_sc as plsc

N = 10000
E = 320000
D = 128
H = 128
DEC = 64
NUM_MASK = 5000

NC = 2
NS = 16
NW = NC * NS
N_PAD = 10240
EPW = E // NW
CH = 80
N_CH = EPW // CH
ROWS_PT = N_PAD // NS
MASK_PAD = 5120
MPW = MASK_PAD // NW

GCH = 80
GK = 2
EPW_PAD = 10240
NCHW = EPW_PAD // GCH
KPRE = 32
NSUP = NCHW // KPRE
NGI = KPRE // GK

_mesh = lambda: plsc.VectorSubcoreMesh(core_axis_name="c", subcore_axis_name="s")


def _fill_1d(ref, n, value):
    def body(i, _):
        ref[pl.ds(i * 16, 16)] = jnp.full((16,), value, jnp.float32)
        return 0
    lax.fori_loop(0, n // 16, body, 0)


def _fill_2d(ref, rows, cols, value):
    def body(i, _):
        ref[i // (cols // 16), pl.ds((i % (cols // 16)) * 16, 16)] = (
            jnp.full((16,), value, jnp.float32))
        return 0
    lax.fori_loop(0, rows * (cols // 16), body, 0)


def _deg_flag_sc(dst, mask_pad):
    @functools.partial(
        pl.kernel,
        out_type=(
            jax.ShapeDtypeStruct((NC, N_PAD), jnp.float32),
            jax.ShapeDtypeStruct((NC, N_PAD), jnp.float32),
        ),
        mesh=_mesh(),
        scratch_types=[
            pltpu.VMEM((CH,), jnp.int32),
            pltpu.VMEM((CH,), jnp.float32),
            pltpu.VMEM((ROWS_PT,), jnp.float32),
            pltpu.VMEM_SHARED((N_PAD,), jnp.float32),
            pltpu.VMEM_SHARED((N_PAD,), jnp.float32),
        ],
    )
    def k(dst_hbm, mask_hbm, deg_out, flag_out, idx_v, ones_v, zero_v,
          deg_acc, flag_acc):
        c = lax.axis_index("c")
        s = lax.axis_index("s")
        wid = s * NC + c

        _fill_1d(ones_v, CH, 1.0)
        _fill_1d(zero_v, ROWS_PT, 0.0)
        pltpu.sync_copy(zero_v, deg_acc.at[pl.ds(s * ROWS_PT, ROWS_PT)])
        pltpu.sync_copy(zero_v, flag_acc.at[pl.ds(s * ROWS_PT, ROWS_PT)])
        plsc.subcore_barrier()

        def ebody(i, _):
            base = wid * EPW + i * CH
            pltpu.sync_copy(dst_hbm.at[pl.ds(base, CH)], idx_v)
            pltpu.sync_copy(ones_v, deg_acc.at[idx_v], add=True)
            return 0
        lax.fori_loop(0, N_CH, ebody, 0)

        def mbody(i, _):
            base = wid * MPW + i * CH
            pltpu.sync_copy(mask_hbm.at[pl.ds(base, CH)], idx_v)
            pltpu.sync_copy(ones_v, flag_acc.at[idx_v], add=True)
            return 0
        lax.fori_loop(0, MPW // CH, mbody, 0)
        plsc.subcore_barrier()

        base = s * ROWS_PT
        pltpu.sync_copy(deg_acc.at[pl.ds(base, ROWS_PT)],
                        deg_out.at[c, pl.ds(base, ROWS_PT)])
        pltpu.sync_copy(flag_acc.at[pl.ds(base, ROWS_PT)],
                        flag_out.at[c, pl.ds(base, ROWS_PT)])

    return k(dst, mask_pad)


def _agg_sc(h, src, dst, width):
    @functools.partial(
        pl.kernel,
        out_type=jax.ShapeDtypeStruct((NC, N_PAD, width), jnp.float32),
        mesh=_mesh(),
        scratch_types=[
            pltpu.VMEM((KPRE, GCH), jnp.int32),
            pltpu.VMEM((KPRE, GCH), jnp.int32),
            [pltpu.VMEM((GCH, width), jnp.float32) for _ in range(GK)],
            [pltpu.SemaphoreType.DMA for _ in range(GK)],
            pltpu.SemaphoreType.DMA,
            pltpu.VMEM_SHARED((N_PAD, width), jnp.float32),
        ],
        compiler_params=pltpu.CompilerParams(
            use_tc_tiling_on_sc=(width % 128 == 0)),
    )
    def k(h_hbm, src_hbm, dst_hbm, out_hbm, sidx, didx, rows,
          gsem, ssem, acc):
        c = lax.axis_index("c")
        s = lax.axis_index("s")
        wid = s * NC + c

        _fill_2d(rows[0], GCH, width, 0.0)
        for kk in range(ROWS_PT // GCH):
            pltpu.sync_copy(rows[0],
                            acc.at[pl.ds(s * ROWS_PT + kk * GCH, GCH), :])
        plsc.subcore_barrier()

        def sbody(sj, _):
            row0 = wid * NCHW + sj * KPRE
            pltpu.sync_copy(src_hbm.at[pl.ds(row0, KPRE), :], sidx)
            pltpu.sync_copy(dst_hbm.at[pl.ds(row0, KPRE), :], didx)

            def gbody(j, _):
                ch = j * GK
                gd = [pltpu.async_copy(h_hbm.at[sidx.at[ch + kk]], rows[kk],
                                       gsem[kk]) for kk in range(GK)]
                sd = []
                for kk in range(GK):
                    gd[kk].wait()
                    sd.append(pltpu.async_copy(rows[kk],
                                               acc.at[didx.at[ch + kk]],
                                               ssem, add=True))
                for kk in range(GK):
                    sd[kk].wait()
                return 0
            lax.fori_loop(0, NGI, gbody, 0)
            return 0
        lax.fori_loop(0, NSUP, sbody, 0)
        plsc.subcore_barrier()

        for kk in range(ROWS_PT // GCH):
            base = s * ROWS_PT + kk * GCH
            pltpu.sync_copy(acc.at[pl.ds(base, GCH), :],
                            out_hbm.at[c, pl.ds(base, GCH), :])

    return k(h, src, dst)


_BLK = 2000
_GRID = N // _BLK
_P = jax.lax.Precision.HIGHEST


def _enc_tc(x, W_enc, mask_token, degt, flagt):
    def body(x_ref, w_ref, mt_ref, degt_ref, flagt_ref, o_ref):
        deg = degt_ref[:, 0:1] + degt_ref[:, 1:2] + 1.0
        dinv = lax.rsqrt(deg)
        flag = flagt_ref[:, 0:1] + flagt_ref[:, 1:2]
        h = jnp.dot(x_ref[...], w_ref[...], precision=_P,
                    preferred_element_type=jnp.float32)
        m1 = jnp.dot(mt_ref[...], w_ref[...], precision=_P,
                     preferred_element_type=jnp.float32)
        o_ref[...] = dinv * jnp.where(flag > 0.0, m1, h)

    return pl.pallas_call(
        body,
        grid=(_GRID,),
        in_specs=[
            pl.BlockSpec((_BLK, D), lambda i: (i, 0)),
            pl.BlockSpec((D, H), lambda i: (0, 0)),
            pl.BlockSpec((1, D), lambda i: (0, 0)),
            pl.BlockSpec((_BLK, NC), lambda i: (i, 0)),
            pl.BlockSpec((_BLK, NC), lambda i: (i, 0)),
        ],
        out_specs=pl.BlockSpec((_BLK, H), lambda i: (i, 0)),
        out_shape=jax.ShapeDtypeStruct((N, H), jnp.float32),
    )(x, W_enc, mask_token, degt, flagt)


def _dec_tc(agg1, h1p, degt, b_enc, W_dec):
    def body(p_ref, h_ref, degt_ref, b_ref, w_ref, o_ref):
        deg = degt_ref[:, 0:1] + degt_ref[:, 1:2] + 1.0
        dinv = lax.rsqrt(deg)
        z = dinv * (p_ref[0] + p_ref[1] + h_ref[...]) + b_ref[...]
        o_ref[...] = dinv * jnp.dot(z, w_ref[...], precision=_P,
                                    preferred_element_type=jnp.float32)

    return pl.pallas_call(
        body,
        grid=(_GRID,),
        in_specs=[
            pl.BlockSpec((NC, _BLK, H), lambda i: (0, i, 0)),
            pl.BlockSpec((_BLK, H), lambda i: (i, 0)),
            pl.BlockSpec((_BLK, NC), lambda i: (i, 0)),
            pl.BlockSpec((1, H), lambda i: (0, 0)),
            pl.BlockSpec((H, DEC), lambda i: (0, 0)),
        ],
        out_specs=pl.BlockSpec((_BLK, DEC), lambda i: (i, 0)),
        out_shape=jax.ShapeDtypeStruct((N, DEC), jnp.float32),
    )(agg1, h1p, degt, b_enc, W_dec)


def _out_tc(agg2, h2p, degt, b_dec, W_mlp, b_mlp):
    def body(p_ref, h_ref, degt_ref, bd_ref, w_ref, bm_ref, o_ref):
        deg = degt_ref[:, 0:1] + degt_ref[:, 1:2] + 1.0
        dinv = lax.rsqrt(deg)
        hdec = jnp.maximum(
            dinv * (p_ref[0] + p_ref[1] + h_ref[...]) + bd_ref[...], 0.0)
        o_ref[...] = jnp.dot(hdec, w_ref[...], precision=_P,
                             preferred_element_type=jnp.float32) + bm_ref[...]

    return pl.pallas_call(
        body,
        grid=(_GRID,),
        in_specs=[
            pl.BlockSpec((NC, _BLK, DEC), lambda i: (0, i, 0)),
            pl.BlockSpec((_BLK, DEC), lambda i: (i, 0)),
            pl.BlockSpec((_BLK, NC), lambda i: (i, 0)),
            pl.BlockSpec((1, DEC), lambda i: (0, 0)),
            pl.BlockSpec((DEC, D), lambda i: (0, 0)),
            pl.BlockSpec((1, D), lambda i: (0, 0)),
        ],
        out_specs=pl.BlockSpec((_BLK, D), lambda i: (i, 0)),
        out_shape=jax.ShapeDtypeStruct((N, D), jnp.float32),
    )(agg2, h2p, degt, b_dec, W_mlp, b_mlp)


def kernel(x, edge_index, mask_indices, mask_token, W_enc, b_enc, W_dec,
           b_dec, W_mlp, b_mlp):
    mi = mask_indices.astype(jnp.int32)
    mask_pad = jnp.concatenate(
        [mi, jnp.full((MASK_PAD - NUM_MASK,), N, jnp.int32)])

    src = edge_index[0]
    dst = edge_index[1]
    pad_n = EPW_PAD - EPW
    src_p = jnp.concatenate(
        [src.reshape(NW, EPW),
         jnp.zeros((NW, pad_n), jnp.int32)], axis=1).reshape(-1, GCH)
    dst_p = jnp.concatenate(
        [dst.reshape(NW, EPW),
         jnp.full((NW, pad_n), N, jnp.int32)], axis=1).reshape(-1, GCH)
    deg_p, flag_p = _deg_flag_sc(dst, mask_pad)
    degt = deg_p[:, :N].T
    flagt = flag_p[:, :N].T

    h1p = _enc_tc(x, W_enc, mask_token, degt, flagt)
    agg1 = _agg_sc(h1p, src_p, dst_p, H)
    h2p = _dec_tc(agg1, h1p, degt, b_enc.reshape(1, H), W_dec)
    agg2 = _agg_sc(h2p, src_p, dst_p, DEC)
    x_rec = _out_tc(agg2, h2p, degt, b_dec.reshape(1, DEC), W_mlp,
                    b_mlp.reshape(1, D))
    return (x_rec, x, mask_indices)


# speedup vs baseline: 1.8171x; 1.8171x over previous
import functools

import jax
import jax.numpy as jnp
from jax import lax
from jax.experimental import pallas as pl
from jax.experimental.pallas import tpu as pltpu
from jax.experimental.pallas import tpu_sc as plsc

N = 10000
E = 320000
D = 128
H = 128
DEC = 64
NUM_MASK = 5000

NC = 2
NS = 16
NW = NC * NS
N_PAD = 10240
EPW = E // NW
CH = 80
N_CH = EPW // CH
ROWS_PT = N_PAD // NS
MASK_PAD = 5120
MPW = MASK_PAD // NW

GCH = 80
GK = 2
NG = N_CH // GK

_mesh = lambda: plsc.VectorSubcoreMesh(core_axis_name="c", subcore_axis_name="s")


def _fill_1d(ref, n, value):
    def body(i, _):
        ref[pl.ds(i * 16, 16)] = jnp.full((16,), value, jnp.float32)
        return 0
    lax.fori_loop(0, n // 16, body, 0)


def _fill_2d(ref, rows, cols, value):
    def body(i, _):
        ref[i // (cols // 16), pl.ds((i % (cols // 16)) * 16, 16)] = (
            jnp.full((16,), value, jnp.float32))
        return 0
    lax.fori_loop(0, rows * (cols // 16), body, 0)


def _deg_flag_sc(dst, mask_pad):
    @functools.partial(
        pl.kernel,
        out_type=(
            jax.ShapeDtypeStruct((NC, N_PAD), jnp.float32),
            jax.ShapeDtypeStruct((NC, N_PAD), jnp.float32),
        ),
        mesh=_mesh(),
        scratch_types=[
            pltpu.VMEM((CH,), jnp.int32),
            pltpu.VMEM((CH,), jnp.float32),
            pltpu.VMEM((ROWS_PT,), jnp.float32),
            pltpu.VMEM_SHARED((N_PAD,), jnp.float32),
            pltpu.VMEM_SHARED((N_PAD,), jnp.float32),
        ],
    )
    def k(dst_hbm, mask_hbm, deg_out, flag_out, idx_v, ones_v, zero_v,
          deg_acc, flag_acc):
        c = lax.axis_index("c")
        s = lax.axis_index("s")
        wid = s * NC + c

        _fill_1d(ones_v, CH, 1.0)
        _fill_1d(zero_v, ROWS_PT, 0.0)
        pltpu.sync_copy(zero_v, deg_acc.at[pl.ds(s * ROWS_PT, ROWS_PT)])
        pltpu.sync_copy(zero_v, flag_acc.at[pl.ds(s * ROWS_PT, ROWS_PT)])
        plsc.subcore_barrier()

        def ebody(i, _):
            base = wid * EPW + i * CH
            pltpu.sync_copy(dst_hbm.at[pl.ds(base, CH)], idx_v)
            pltpu.sync_copy(ones_v, deg_acc.at[idx_v], add=True)
            return 0
        lax.fori_loop(0, N_CH, ebody, 0)

        def mbody(i, _):
            base = wid * MPW + i * CH
            pltpu.sync_copy(mask_hbm.at[pl.ds(base, CH)], idx_v)
            pltpu.sync_copy(ones_v, flag_acc.at[idx_v], add=True)
            return 0
        lax.fori_loop(0, MPW // CH, mbody, 0)
        plsc.subcore_barrier()

        base = s * ROWS_PT
        pltpu.sync_copy(deg_acc.at[pl.ds(base, ROWS_PT)],
                        deg_out.at[c, pl.ds(base, ROWS_PT)])
        pltpu.sync_copy(flag_acc.at[pl.ds(base, ROWS_PT)],
                        flag_out.at[c, pl.ds(base, ROWS_PT)])

    return k(dst, mask_pad)


def _agg_sc(h, src, dst, width):
    @functools.partial(
        pl.kernel,
        out_type=jax.ShapeDtypeStruct((NC, N_PAD, width), jnp.float32),
        mesh=_mesh(),
        scratch_types=[
            [pltpu.VMEM((GCH,), jnp.int32) for _ in range(GK)],
            [pltpu.VMEM((GCH,), jnp.int32) for _ in range(GK)],
            [pltpu.VMEM((GCH, width), jnp.float32) for _ in range(GK)],
            [pltpu.SemaphoreType.DMA for _ in range(GK)],
            [pltpu.SemaphoreType.DMA for _ in range(GK)],
            [pltpu.SemaphoreType.DMA for _ in range(GK)],
            pltpu.SemaphoreType.DMA,
            pltpu.VMEM_SHARED((N_PAD, width), jnp.float32),
        ],
        compiler_params=pltpu.CompilerParams(
            use_tc_tiling_on_sc=(width % 128 == 0)),
    )
    def k(h_hbm, src_hbm, dst_hbm, out_hbm, sidx, didx, rows,
          gsem, isems, isemd, ssem, acc):
        c = lax.axis_index("c")
        s = lax.axis_index("s")
        wid = s * NC + c
        wbase = wid * EPW

        _fill_2d(rows[0], GCH, width, 0.0)
        for kk in range(ROWS_PT // GCH):
            pltpu.sync_copy(rows[0],
                            acc.at[pl.ds(s * ROWS_PT + kk * GCH, GCH), :])
        plsc.subcore_barrier()

        def chunk_work(base, nk):
            sd = [pltpu.async_copy(src_hbm.at[pl.ds(base + kk * GCH, GCH)],
                                   sidx[kk], isems[kk]) for kk in range(nk)]
            dd = [pltpu.async_copy(dst_hbm.at[pl.ds(base + kk * GCH, GCH)],
                                   didx[kk], isemd[kk]) for kk in range(nk)]
            gd = []
            for kk in range(nk):
                sd[kk].wait()
                gd.append(pltpu.async_copy(h_hbm.at[sidx[kk]], rows[kk],
                                           gsem[kk]))
            for kk in range(nk):
                dd[kk].wait()
            ss = []
            for kk in range(nk):
                gd[kk].wait()
                ss.append(pltpu.async_copy(rows[kk], acc.at[didx[kk]],
                                           ssem, add=True))
            for kk in range(nk):
                ss[kk].wait()

        def gbody(j, _):
            chunk_work(wbase + j * GK * GCH, GK)
            return 0
        lax.fori_loop(0, NG, gbody, 0)
        chunk_work(wbase + NG * GK * GCH, N_CH - NG * GK)
        plsc.subcore_barrier()

        for kk in range(ROWS_PT // GCH):
            base = s * ROWS_PT + kk * GCH
            pltpu.sync_copy(acc.at[pl.ds(base, GCH), :],
                            out_hbm.at[c, pl.ds(base, GCH), :])

    return k(h, src, dst)


_BLK = 2000
_GRID = N // _BLK
_P = jax.lax.Precision.HIGHEST


def _enc_tc(x, W_enc, mask_token, degt, flagt):
    def body(x_ref, w_ref, mt_ref, degt_ref, flagt_ref, o_ref):
        deg = degt_ref[:, 0:1] + degt_ref[:, 1:2] + 1.0
        dinv = lax.rsqrt(deg)
        flag = flagt_ref[:, 0:1] + flagt_ref[:, 1:2]
        h = jnp.dot(x_ref[...], w_ref[...], precision=_P,
                    preferred_element_type=jnp.float32)
        m1 = jnp.dot(mt_ref[...], w_ref[...], precision=_P,
                     preferred_element_type=jnp.float32)
        o_ref[...] = dinv * jnp.where(flag > 0.0, m1, h)

    return pl.pallas_call(
        body,
        grid=(_GRID,),
        in_specs=[
            pl.BlockSpec((_BLK, D), lambda i: (i, 0)),
            pl.BlockSpec((D, H), lambda i: (0, 0)),
            pl.BlockSpec((1, D), lambda i: (0, 0)),
            pl.BlockSpec((_BLK, NC), lambda i: (i, 0)),
            pl.BlockSpec((_BLK, NC), lambda i: (i, 0)),
        ],
        out_specs=pl.BlockSpec((_BLK, H), lambda i: (i, 0)),
        out_shape=jax.ShapeDtypeStruct((N, H), jnp.float32),
    )(x, W_enc, mask_token, degt, flagt)


def _dec_tc(agg1, h1p, degt, b_enc, W_dec):
    def body(p_ref, h_ref, degt_ref, b_ref, w_ref, o_ref):
        deg = degt_ref[:, 0:1] + degt_ref[:, 1:2] + 1.0
        dinv = lax.rsqrt(deg)
        z = dinv * (p_ref[0] + p_ref[1] + h_ref[...]) + b_ref[...]
        o_ref[...] = dinv * jnp.dot(z, w_ref[...], precision=_P,
                                    preferred_element_type=jnp.float32)

    return pl.pallas_call(
        body,
        grid=(_GRID,),
        in_specs=[
            pl.BlockSpec((NC, _BLK, H), lambda i: (0, i, 0)),
            pl.BlockSpec((_BLK, H), lambda i: (i, 0)),
            pl.BlockSpec((_BLK, NC), lambda i: (i, 0)),
            pl.BlockSpec((1, H), lambda i: (0, 0)),
            pl.BlockSpec((H, DEC), lambda i: (0, 0)),
        ],
        out_specs=pl.BlockSpec((_BLK, DEC), lambda i: (i, 0)),
        out_shape=jax.ShapeDtypeStruct((N, DEC), jnp.float32),
    )(agg1, h1p, degt, b_enc, W_dec)


def _out_tc(agg2, h2p, degt, b_dec, W_mlp, b_mlp):
    def body(p_ref, h_ref, degt_ref, bd_ref, w_ref, bm_ref, o_ref):
        deg = degt_ref[:, 0:1] + degt_ref[:, 1:2] + 1.0
        dinv = lax.rsqrt(deg)
        hdec = jnp.maximum(
            dinv * (p_ref[0] + p_ref[1] + h_ref[...]) + bd_ref[...], 0.0)
        o_ref[...] = jnp.dot(hdec, w_ref[...], precision=_P,
                             preferred_element_type=jnp.float32) + bm_ref[...]

    return pl.pallas_call(
        body,
        grid=(_GRID,),
        in_specs=[
            pl.BlockSpec((NC, _BLK, DEC), lambda i: (0, i, 0)),
            pl.BlockSpec((_BLK, DEC), lambda i: (i, 0)),
            pl.BlockSpec((_BLK, NC), lambda i: (i, 0)),
            pl.BlockSpec((1, DEC), lambda i: (0, 0)),
            pl.BlockSpec((DEC, D), lambda i: (0, 0)),
            pl.BlockSpec((1, D), lambda i: (0, 0)),
        ],
        out_specs=pl.BlockSpec((_BLK, D), lambda i: (i, 0)),
        out_shape=jax.ShapeDtypeStruct((N, D), jnp.float32),
    )(agg2, h2p, degt, b_dec, W_mlp, b_mlp)


def kernel(x, edge_index, mask_indices, mask_token, W_enc, b_enc, W_dec,
           b_dec, W_mlp, b_mlp):
    mi = mask_indices.astype(jnp.int32)
    mask_pad = jnp.concatenate(
        [mi, jnp.full((MASK_PAD - NUM_MASK,), N, jnp.int32)])

    src = edge_index[0]
    dst = edge_index[1]
    deg_p, flag_p = _deg_flag_sc(dst, mask_pad)
    degt = deg_p[:, :N].T
    flagt = flag_p[:, :N].T

    h1p = _enc_tc(x, W_enc, mask_token, degt, flagt)
    agg1 = _agg_sc(h1p, src, dst, H)
    h2p = _dec_tc(agg1, h1p, degt, b_enc.reshape(1, H), W_dec)
    agg2 = _agg_sc(h2p, src, dst, DEC)
    x_rec = _out_tc(agg2, h2p, degt, b_dec.reshape(1, DEC), W_mlp,
                    b_mlp.reshape(1, D))
    return (x_rec, x, mask_indices)


# speedup vs baseline: 2.0077x; 1.1049x over previous
import functools

import jax
import jax.numpy as jnp
from jax import lax
from jax.experimental import pallas as pl
from jax.experimental.pallas import tpu as pltpu
from jax.experimental.pallas import tpu_sc as plsc

N = 10000
E = 320000
D = 128
H = 128
DEC = 64
NUM_MASK = 5000

NC = 2
NS = 16
NW = NC * NS
N_PAD = 10240
EPW = E // NW
CH = 80
N_CH = EPW // CH
ROWS_PT = N_PAD // NS
MASK_PAD = 5120
MPW = MASK_PAD // NW

GCH = 80
GK = 2
NG = N_CH // GK

_mesh = lambda: plsc.VectorSubcoreMesh(core_axis_name="c", subcore_axis_name="s")


def _fill_1d(ref, n, value):
    def body(i, _):
        ref[pl.ds(i * 16, 16)] = jnp.full((16,), value, jnp.float32)
        return 0
    lax.fori_loop(0, n // 16, body, 0)


def _fill_2d(ref, rows, cols, value):
    def body(i, _):
        ref[i // (cols // 16), pl.ds((i % (cols // 16)) * 16, 16)] = (
            jnp.full((16,), value, jnp.float32))
        return 0
    lax.fori_loop(0, rows * (cols // 16), body, 0)


def _deg_flag_sc(dst, mask_pad):
    @functools.partial(
        pl.kernel,
        out_type=(
            jax.ShapeDtypeStruct((NC, N_PAD), jnp.float32),
            jax.ShapeDtypeStruct((NC, N_PAD), jnp.float32),
        ),
        mesh=_mesh(),
        scratch_types=[
            [pltpu.VMEM((CH,), jnp.int32) for _ in range(4)],
            pltpu.VMEM((CH,), jnp.float32),
            pltpu.VMEM((ROWS_PT,), jnp.float32),
            [pltpu.SemaphoreType.DMA for _ in range(4)],
            pltpu.SemaphoreType.DMA,
            pltpu.VMEM_SHARED((N_PAD,), jnp.float32),
            pltpu.VMEM_SHARED((N_PAD,), jnp.float32),
        ],
    )
    def k(dst_hbm, mask_hbm, deg_out, flag_out, idx, ones_v, zero_v,
          isems, ssem, deg_acc, flag_acc):
        c = lax.axis_index("c")
        s = lax.axis_index("s")
        wid = s * NC + c

        _fill_1d(ones_v, CH, 1.0)
        _fill_1d(zero_v, ROWS_PT, 0.0)
        pltpu.sync_copy(zero_v, deg_acc.at[pl.ds(s * ROWS_PT, ROWS_PT)])
        pltpu.sync_copy(zero_v, flag_acc.at[pl.ds(s * ROWS_PT, ROWS_PT)])
        plsc.subcore_barrier()

        def hist_chunks(src_idx_hbm, base, nk, acc):
            idd = [pltpu.async_copy(src_idx_hbm.at[pl.ds(base + kk * CH, CH)],
                                    idx[kk], isems[kk]) for kk in range(nk)]
            ss = []
            for kk in range(nk):
                idd[kk].wait()
                ss.append(pltpu.async_copy(ones_v, acc.at[idx[kk]], ssem,
                                           add=True))
            for kk in range(nk):
                ss[kk].wait()

        def ebody(i, _):
            hist_chunks(dst_hbm, wid * EPW + i * 4 * CH, 4, deg_acc)
            return 0
        lax.fori_loop(0, N_CH // 4, ebody, 0)
        hist_chunks(dst_hbm, wid * EPW + (N_CH // 4) * 4 * CH, N_CH % 4,
                    deg_acc)
        hist_chunks(mask_hbm, wid * MPW, MPW // CH, flag_acc)
        plsc.subcore_barrier()

        base = s * ROWS_PT
        pltpu.sync_copy(deg_acc.at[pl.ds(base, ROWS_PT)],
                        deg_out.at[c, pl.ds(base, ROWS_PT)])
        pltpu.sync_copy(flag_acc.at[pl.ds(base, ROWS_PT)],
                        flag_out.at[c, pl.ds(base, ROWS_PT)])

    return k(dst, mask_pad)


def _agg_sc(h, src, dst, width):
    @functools.partial(
        pl.kernel,
        out_type=jax.ShapeDtypeStruct((NC, N_PAD, width), jnp.float32),
        mesh=_mesh(),
        scratch_types=[
            [pltpu.VMEM((GCH,), jnp.int32) for _ in range(GK)],
            [pltpu.VMEM((GCH,), jnp.int32) for _ in range(GK)],
            [pltpu.VMEM((GCH, width), jnp.float32) for _ in range(GK)],
            [pltpu.SemaphoreType.DMA for _ in range(GK)],
            [pltpu.SemaphoreType.DMA for _ in range(GK)],
            [pltpu.SemaphoreType.DMA for _ in range(GK)],
            pltpu.SemaphoreType.DMA,
            pltpu.VMEM_SHARED((N_PAD, width), jnp.float32),
        ],
        compiler_params=pltpu.CompilerParams(
            use_tc_tiling_on_sc=(width % 128 == 0)),
    )
    def k(h_hbm, src_hbm, dst_hbm, out_hbm, sidx, didx, rows,
          gsem, isems, isemd, ssem, acc):
        c = lax.axis_index("c")
        s = lax.axis_index("s")
        wid = s * NC + c
        wbase = wid * EPW

        _fill_2d(rows[0], GCH, width, 0.0)
        for kk in range(ROWS_PT // GCH):
            pltpu.sync_copy(rows[0],
                            acc.at[pl.ds(s * ROWS_PT + kk * GCH, GCH), :])
        plsc.subcore_barrier()

        def chunk_work(base, nk):
            sd = [pltpu.async_copy(src_hbm.at[pl.ds(base + kk * GCH, GCH)],
                                   sidx[kk], isems[kk]) for kk in range(nk)]
            dd = [pltpu.async_copy(dst_hbm.at[pl.ds(base + kk * GCH, GCH)],
                                   didx[kk], isemd[kk]) for kk in range(nk)]
            gd = []
            for kk in range(nk):
                sd[kk].wait()
                gd.append(pltpu.async_copy(h_hbm.at[sidx[kk]], rows[kk],
                                           gsem[kk]))
            for kk in range(nk):
                dd[kk].wait()
            ss = []
            for kk in range(nk):
                gd[kk].wait()
                ss.append(pltpu.async_copy(rows[kk], acc.at[didx[kk]],
                                           ssem, add=True))
            for kk in range(nk):
                ss[kk].wait()

        def gbody(j, _):
            chunk_work(wbase + j * GK * GCH, GK)
            return 0
        lax.fori_loop(0, NG, gbody, 0)
        chunk_work(wbase + NG * GK * GCH, N_CH - NG * GK)
        plsc.subcore_barrier()

        for kk in range(ROWS_PT // GCH):
            base = s * ROWS_PT + kk * GCH
            pltpu.sync_copy(acc.at[pl.ds(base, GCH), :],
                            out_hbm.at[c, pl.ds(base, GCH), :])

    return k(h, src, dst)


_BLK = 2000
_GRID = N // _BLK
_P = jax.lax.Precision.HIGHEST


def _enc_tc(x, W_enc, mask_token, degt, flagt):
    def body(x_ref, w_ref, mt_ref, degt_ref, flagt_ref, o_ref):
        deg = degt_ref[:, 0:1] + degt_ref[:, 1:2] + 1.0
        dinv = lax.rsqrt(deg)
        flag = flagt_ref[:, 0:1] + flagt_ref[:, 1:2]
        h = jnp.dot(x_ref[...], w_ref[...], precision=_P,
                    preferred_element_type=jnp.float32)
        m1 = jnp.dot(mt_ref[...], w_ref[...], precision=_P,
                     preferred_element_type=jnp.float32)
        o_ref[...] = dinv * jnp.where(flag > 0.0, m1, h)

    return pl.pallas_call(
        body,
        grid=(_GRID,),
        in_specs=[
            pl.BlockSpec((_BLK, D), lambda i: (i, 0)),
            pl.BlockSpec((D, H), lambda i: (0, 0)),
            pl.BlockSpec((1, D), lambda i: (0, 0)),
            pl.BlockSpec((_BLK, NC), lambda i: (i, 0)),
            pl.BlockSpec((_BLK, NC), lambda i: (i, 0)),
        ],
        out_specs=pl.BlockSpec((_BLK, H), lambda i: (i, 0)),
        out_shape=jax.ShapeDtypeStruct((N, H), jnp.float32),
    )(x, W_enc, mask_token, degt, flagt)


def _dec_tc(agg1, h1p, degt, b_enc, W_dec):
    def body(p_ref, h_ref, degt_ref, b_ref, w_ref, o_ref):
        deg = degt_ref[:, 0:1] + degt_ref[:, 1:2] + 1.0
        dinv = lax.rsqrt(deg)
        z = dinv * (p_ref[0] + p_ref[1] + h_ref[...]) + b_ref[...]
        o_ref[...] = dinv * jnp.dot(z, w_ref[...], precision=_P,
                                    preferred_element_type=jnp.float32)

    return pl.pallas_call(
        body,
        grid=(_GRID,),
        in_specs=[
            pl.BlockSpec((NC, _BLK, H), lambda i: (0, i, 0)),
            pl.BlockSpec((_BLK, H), lambda i: (i, 0)),
            pl.BlockSpec((_BLK, NC), lambda i: (i, 0)),
            pl.BlockSpec((1, H), lambda i: (0, 0)),
            pl.BlockSpec((H, DEC), lambda i: (0, 0)),
        ],
        out_specs=pl.BlockSpec((_BLK, DEC), lambda i: (i, 0)),
        out_shape=jax.ShapeDtypeStruct((N, DEC), jnp.float32),
    )(agg1, h1p, degt, b_enc, W_dec)


def _out_tc(agg2, h2p, degt, b_dec, W_mlp, b_mlp):
    def body(p_ref, h_ref, degt_ref, bd_ref, w_ref, bm_ref, o_ref):
        deg = degt_ref[:, 0:1] + degt_ref[:, 1:2] + 1.0
        dinv = lax.rsqrt(deg)
        hdec = jnp.maximum(
            dinv * (p_ref[0] + p_ref[1] + h_ref[...]) + bd_ref[...], 0.0)
        o_ref[...] = jnp.dot(hdec, w_ref[...], precision=_P,
                             preferred_element_type=jnp.float32) + bm_ref[...]

    return pl.pallas_call(
        body,
        grid=(_GRID,),
        in_specs=[
            pl.BlockSpec((NC, _BLK, DEC), lambda i: (0, i, 0)),
            pl.BlockSpec((_BLK, DEC), lambda i: (i, 0)),
            pl.BlockSpec((_BLK, NC), lambda i: (i, 0)),
            pl.BlockSpec((1, DEC), lambda i: (0, 0)),
            pl.BlockSpec((DEC, D), lambda i: (0, 0)),
            pl.BlockSpec((1, D), lambda i: (0, 0)),
        ],
        out_specs=pl.BlockSpec((_BLK, D), lambda i: (i, 0)),
        out_shape=jax.ShapeDtypeStruct((N, D), jnp.float32),
    )(agg2, h2p, degt, b_dec, W_mlp, b_mlp)


def kernel(x, edge_index, mask_indices, mask_token, W_enc, b_enc, W_dec,
           b_dec, W_mlp, b_mlp):
    mi = mask_indices.astype(jnp.int32)
    mask_pad = jnp.concatenate(
        [mi, jnp.full((MASK_PAD - NUM_MASK,), N, jnp.int32)])

    src = edge_index[0]
    dst = edge_index[1]
    deg_p, flag_p = _deg_flag_sc(dst, mask_pad)
    degt = deg_p[:, :N].T
    flagt = flag_p[:, :N].T

    h1p = _enc_tc(x, W_enc, mask_token, degt, flagt)
    agg1 = _agg_sc(h1p, src, dst, H)
    h2p = _dec_tc(agg1, h1p, degt, b_enc.reshape(1, H), W_dec)
    agg2 = _agg_sc(h2p, src, dst, DEC)
    x_rec = _out_tc(agg2, h2p, degt, b_dec.reshape(1, DEC), W_mlp,
                    b_mlp.reshape(1, D))
    return (x_rec, x, mask_indices)


# speedup vs baseline: 2.2190x; 1.1053x over previous
import functools

import jax
import jax.numpy as jnp
from jax import lax
from jax.experimental import pallas as pl
from jax.experimental.pallas import tpu as pltpu
from jax.experimental.pallas import tpu_sc as plsc

N = 10000
E = 320000
D = 128
H = 128
DEC = 64
NUM_MASK = 5000

NC = 2
NS = 16
NW = NC * NS
N_PAD = 10240
EPW = E // NW
CH = 80
N_CH = EPW // CH
ROWS_PT = N_PAD // NS
MASK_PAD = 5120
MPW = MASK_PAD // NW

GCH = 80
GK = 3
NG = N_CH // GK

_mesh = lambda: plsc.VectorSubcoreMesh(core_axis_name="c", subcore_axis_name="s")


def _fill_1d(ref, n, value):
    def body(i, _):
        ref[pl.ds(i * 16, 16)] = jnp.full((16,), value, jnp.float32)
        return 0
    lax.fori_loop(0, n // 16, body, 0)


def _fill_2d(ref, rows, cols, value):
    def body(i, _):
        ref[i // (cols // 16), pl.ds((i % (cols // 16)) * 16, 16)] = (
            jnp.full((16,), value, jnp.float32))
        return 0
    lax.fori_loop(0, rows * (cols // 16), body, 0)


def _deg_flag_sc(dst, mask_pad):
    @functools.partial(
        pl.kernel,
        out_type=(
            jax.ShapeDtypeStruct((NC, N_PAD), jnp.float32),
            jax.ShapeDtypeStruct((NC, N_PAD), jnp.float32),
        ),
        mesh=_mesh(),
        scratch_types=[
            [pltpu.VMEM((CH,), jnp.int32) for _ in range(4)],
            pltpu.VMEM((CH,), jnp.float32),
            pltpu.VMEM((ROWS_PT,), jnp.float32),
            [pltpu.SemaphoreType.DMA for _ in range(4)],
            pltpu.SemaphoreType.DMA,
            pltpu.VMEM_SHARED((N_PAD,), jnp.float32),
            pltpu.VMEM_SHARED((N_PAD,), jnp.float32),
        ],
    )
    def k(dst_hbm, mask_hbm, deg_out, flag_out, idx, ones_v, zero_v,
          isems, ssem, deg_acc, flag_acc):
        c = lax.axis_index("c")
        s = lax.axis_index("s")
        wid = s * NC + c

        _fill_1d(ones_v, CH, 1.0)
        _fill_1d(zero_v, ROWS_PT, 0.0)
        pltpu.sync_copy(zero_v, deg_acc.at[pl.ds(s * ROWS_PT, ROWS_PT)])
        pltpu.sync_copy(zero_v, flag_acc.at[pl.ds(s * ROWS_PT, ROWS_PT)])
        plsc.subcore_barrier()

        def hist_chunks(src_idx_hbm, base, nk, acc):
            idd = [pltpu.async_copy(src_idx_hbm.at[pl.ds(base + kk * CH, CH)],
                                    idx[kk], isems[kk]) for kk in range(nk)]
            ss = []
            for kk in range(nk):
                idd[kk].wait()
                ss.append(pltpu.async_copy(ones_v, acc.at[idx[kk]], ssem,
                                           add=True))
            for kk in range(nk):
                ss[kk].wait()

        def ebody(i, _):
            hist_chunks(dst_hbm, wid * EPW + i * 4 * CH, 4, deg_acc)
            return 0
        lax.fori_loop(0, N_CH // 4, ebody, 0)
        hist_chunks(dst_hbm, wid * EPW + (N_CH // 4) * 4 * CH, N_CH % 4,
                    deg_acc)
        hist_chunks(mask_hbm, wid * MPW, MPW // CH, flag_acc)
        plsc.subcore_barrier()

        base = s * ROWS_PT
        pltpu.sync_copy(deg_acc.at[pl.ds(base, ROWS_PT)],
                        deg_out.at[c, pl.ds(base, ROWS_PT)])
        pltpu.sync_copy(flag_acc.at[pl.ds(base, ROWS_PT)],
                        flag_out.at[c, pl.ds(base, ROWS_PT)])

    return k(dst, mask_pad)


def _agg_sc(h, src, dst, width):
    @functools.partial(
        pl.kernel,
        out_type=jax.ShapeDtypeStruct((NC, N_PAD, width), jnp.float32),
        mesh=_mesh(),
        scratch_types=[
            [pltpu.VMEM((GCH,), jnp.int32) for _ in range(GK)],
            [pltpu.VMEM((GCH,), jnp.int32) for _ in range(GK)],
            [pltpu.VMEM((GCH, width), jnp.float32) for _ in range(GK)],
            [pltpu.SemaphoreType.DMA for _ in range(GK)],
            [pltpu.SemaphoreType.DMA for _ in range(GK)],
            [pltpu.SemaphoreType.DMA for _ in range(GK)],
            pltpu.SemaphoreType.DMA,
            pltpu.VMEM_SHARED((N_PAD, width), jnp.float32),
        ],
        compiler_params=pltpu.CompilerParams(
            use_tc_tiling_on_sc=(width % 128 == 0)),
    )
    def k(h_hbm, src_hbm, dst_hbm, out_hbm, sidx, didx, rows,
          gsem, isems, isemd, ssem, acc):
        c = lax.axis_index("c")
        s = lax.axis_index("s")
        wid = s * NC + c
        wbase = wid * EPW

        _fill_2d(rows[0], GCH, width, 0.0)
        for kk in range(ROWS_PT // GCH):
            pltpu.sync_copy(rows[0],
                            acc.at[pl.ds(s * ROWS_PT + kk * GCH, GCH), :])
        plsc.subcore_barrier()

        def chunk_work(base, nk):
            sd = [pltpu.async_copy(src_hbm.at[pl.ds(base + kk * GCH, GCH)],
                                   sidx[kk], isems[kk]) for kk in range(nk)]
            dd = [pltpu.async_copy(dst_hbm.at[pl.ds(base + kk * GCH, GCH)],
                                   didx[kk], isemd[kk]) for kk in range(nk)]
            gd = []
            for kk in range(nk):
                sd[kk].wait()
                gd.append(pltpu.async_copy(h_hbm.at[sidx[kk]], rows[kk],
                                           gsem[kk]))
            for kk in range(nk):
                dd[kk].wait()
            ss = []
            for kk in range(nk):
                gd[kk].wait()
                ss.append(pltpu.async_copy(rows[kk], acc.at[didx[kk]],
                                           ssem, add=True))
            for kk in range(nk):
                ss[kk].wait()

        def gbody(j, _):
            chunk_work(wbase + j * GK * GCH, GK)
            return 0
        lax.fori_loop(0, NG, gbody, 0)
        chunk_work(wbase + NG * GK * GCH, N_CH - NG * GK)
        plsc.subcore_barrier()

        for kk in range(ROWS_PT // GCH):
            base = s * ROWS_PT + kk * GCH
            pltpu.sync_copy(acc.at[pl.ds(base, GCH), :],
                            out_hbm.at[c, pl.ds(base, GCH), :])

    return k(h, src, dst)


_BLK = 2000
_GRID = N // _BLK
_P = jax.lax.Precision.HIGHEST


def _enc_tc(x, W_enc, mask_token, degt, flagt):
    def body(x_ref, w_ref, mt_ref, degt_ref, flagt_ref, o_ref):
        deg = degt_ref[:, 0:1] + degt_ref[:, 1:2] + 1.0
        dinv = lax.rsqrt(deg)
        flag = flagt_ref[:, 0:1] + flagt_ref[:, 1:2]
        h = jnp.dot(x_ref[...], w_ref[...], precision=_P,
                    preferred_element_type=jnp.float32)
        m1 = jnp.dot(mt_ref[...], w_ref[...], precision=_P,
                     preferred_element_type=jnp.float32)
        o_ref[...] = dinv * jnp.where(flag > 0.0, m1, h)

    return pl.pallas_call(
        body,
        grid=(_GRID,),
        in_specs=[
            pl.BlockSpec((_BLK, D), lambda i: (i, 0)),
            pl.BlockSpec((D, H), lambda i: (0, 0)),
            pl.BlockSpec((1, D), lambda i: (0, 0)),
            pl.BlockSpec((_BLK, NC), lambda i: (i, 0)),
            pl.BlockSpec((_BLK, NC), lambda i: (i, 0)),
        ],
        out_specs=pl.BlockSpec((_BLK, H), lambda i: (i, 0)),
        out_shape=jax.ShapeDtypeStruct((N, H), jnp.float32),
    )(x, W_enc, mask_token, degt, flagt)


def _dec_tc(agg1, h1p, degt, b_enc, W_dec):
    def body(p_ref, h_ref, degt_ref, b_ref, w_ref, o_ref):
        deg = degt_ref[:, 0:1] + degt_ref[:, 1:2] + 1.0
        dinv = lax.rsqrt(deg)
        z = dinv * (p_ref[0] + p_ref[1] + h_ref[...]) + b_ref[...]
        o_ref[...] = dinv * jnp.dot(z, w_ref[...], precision=_P,
                                    preferred_element_type=jnp.float32)

    return pl.pallas_call(
        body,
        grid=(_GRID,),
        in_specs=[
            pl.BlockSpec((NC, _BLK, H), lambda i: (0, i, 0)),
            pl.BlockSpec((_BLK, H), lambda i: (i, 0)),
            pl.BlockSpec((_BLK, NC), lambda i: (i, 0)),
            pl.BlockSpec((1, H), lambda i: (0, 0)),
            pl.BlockSpec((H, DEC), lambda i: (0, 0)),
        ],
        out_specs=pl.BlockSpec((_BLK, DEC), lambda i: (i, 0)),
        out_shape=jax.ShapeDtypeStruct((N, DEC), jnp.float32),
    )(agg1, h1p, degt, b_enc, W_dec)


def _out_tc(agg2, h2p, degt, b_dec, W_mlp, b_mlp):
    def body(p_ref, h_ref, degt_ref, bd_ref, w_ref, bm_ref, o_ref):
        deg = degt_ref[:, 0:1] + degt_ref[:, 1:2] + 1.0
        dinv = lax.rsqrt(deg)
        hdec = jnp.maximum(
            dinv * (p_ref[0] + p_ref[1] + h_ref[...]) + bd_ref[...], 0.0)
        o_ref[...] = jnp.dot(hdec, w_ref[...], precision=_P,
                             preferred_element_type=jnp.float32) + bm_ref[...]

    return pl.pallas_call(
        body,
        grid=(_GRID,),
        in_specs=[
            pl.BlockSpec((NC, _BLK, DEC), lambda i: (0, i, 0)),
            pl.BlockSpec((_BLK, DEC), lambda i: (i, 0)),
            pl.BlockSpec((_BLK, NC), lambda i: (i, 0)),
            pl.BlockSpec((1, DEC), lambda i: (0, 0)),
            pl.BlockSpec((DEC, D), lambda i: (0, 0)),
            pl.BlockSpec((1, D), lambda i: (0, 0)),
        ],
        out_specs=pl.BlockSpec((_BLK, D), lambda i: (i, 0)),
        out_shape=jax.ShapeDtypeStruct((N, D), jnp.float32),
    )(agg2, h2p, degt, b_dec, W_mlp, b_mlp)


def kernel(x, edge_index, mask_indices, mask_token, W_enc, b_enc, W_dec,
           b_dec, W_mlp, b_mlp):
    mi = mask_indices.astype(jnp.int32)
    mask_pad = jnp.concatenate(
        [mi, jnp.full((MASK_PAD - NUM_MASK,), N, jnp.int32)])

    src = edge_index[0]
    dst = edge_index[1]
    deg_p, flag_p = _deg_flag_sc(dst, mask_pad)
    degt = deg_p[:, :N].T
    flagt = flag_p[:, :N].T

    h1p = _enc_tc(x, W_enc, mask_token, degt, flagt)
    agg1 = _agg_sc(h1p, src, dst, H)
    h2p = _dec_tc(agg1, h1p, degt, b_enc.reshape(1, H), W_dec)
    agg2 = _agg_sc(h2p, src, dst, DEC)
    x_rec = _out_tc(agg2, h2p, degt, b_dec.reshape(1, DEC), W_mlp,
                    b_mlp.reshape(1, D))
    return (x_rec, x, mask_indices)


# speedup vs baseline: 2.3369x; 1.0532x over previous
import functools

import jax
import jax.numpy as jnp
from jax import lax
from jax.experimental import pallas as pl
from jax.experimental.pallas import tpu as pltpu
from jax.experimental.pallas import tpu_sc as plsc

N = 10000
E = 320000
D = 128
H = 128
DEC = 64
NUM_MASK = 5000

NC = 2
NS = 16
NW = NC * NS
N_PAD = 10240
EPW = E // NW
CH = 80
N_CH = EPW // CH
ROWS_PT = N_PAD // NS
MASK_PAD = 5120
MPW = MASK_PAD // NW

GCH = 80
GK = 4
NG = N_CH // GK

_mesh = lambda: plsc.VectorSubcoreMesh(core_axis_name="c", subcore_axis_name="s")


def _fill_1d(ref, n, value):
    def body(i, _):
        ref[pl.ds(i * 16, 16)] = jnp.full((16,), value, jnp.float32)
        return 0
    lax.fori_loop(0, n // 16, body, 0)


def _fill_2d(ref, rows, cols, value):
    def body(i, _):
        ref[i // (cols // 16), pl.ds((i % (cols // 16)) * 16, 16)] = (
            jnp.full((16,), value, jnp.float32))
        return 0
    lax.fori_loop(0, rows * (cols // 16), body, 0)


def _deg_flag_sc(dst, mask_pad):
    @functools.partial(
        pl.kernel,
        out_type=(
            jax.ShapeDtypeStruct((NC, N_PAD), jnp.float32),
            jax.ShapeDtypeStruct((NC, N_PAD), jnp.float32),
        ),
        mesh=_mesh(),
        scratch_types=[
            [pltpu.VMEM((CH,), jnp.int32) for _ in range(4)],
            pltpu.VMEM((CH,), jnp.float32),
            pltpu.VMEM((ROWS_PT,), jnp.float32),
            [pltpu.SemaphoreType.DMA for _ in range(4)],
            pltpu.SemaphoreType.DMA,
            pltpu.VMEM_SHARED((N_PAD,), jnp.float32),
            pltpu.VMEM_SHARED((N_PAD,), jnp.float32),
        ],
    )
    def k(dst_hbm, mask_hbm, deg_out, flag_out, idx, ones_v, zero_v,
          isems, ssem, deg_acc, flag_acc):
        c = lax.axis_index("c")
        s = lax.axis_index("s")
        wid = s * NC + c

        _fill_1d(ones_v, CH, 1.0)
        _fill_1d(zero_v, ROWS_PT, 0.0)
        pltpu.sync_copy(zero_v, deg_acc.at[pl.ds(s * ROWS_PT, ROWS_PT)])
        pltpu.sync_copy(zero_v, flag_acc.at[pl.ds(s * ROWS_PT, ROWS_PT)])
        plsc.subcore_barrier()

        def hist_chunks(src_idx_hbm, base, nk, acc):
            idd = [pltpu.async_copy(src_idx_hbm.at[pl.ds(base + kk * CH, CH)],
                                    idx[kk], isems[kk]) for kk in range(nk)]
            ss = []
            for kk in range(nk):
                idd[kk].wait()
                ss.append(pltpu.async_copy(ones_v, acc.at[idx[kk]], ssem,
                                           add=True))
            for kk in range(nk):
                ss[kk].wait()

        def ebody(i, _):
            hist_chunks(dst_hbm, wid * EPW + i * 4 * CH, 4, deg_acc)
            return 0
        lax.fori_loop(0, N_CH // 4, ebody, 0)
        hist_chunks(dst_hbm, wid * EPW + (N_CH // 4) * 4 * CH, N_CH % 4,
                    deg_acc)
        hist_chunks(mask_hbm, wid * MPW, MPW // CH, flag_acc)
        plsc.subcore_barrier()

        base = s * ROWS_PT
        pltpu.sync_copy(deg_acc.at[pl.ds(base, ROWS_PT)],
                        deg_out.at[c, pl.ds(base, ROWS_PT)])
        pltpu.sync_copy(flag_acc.at[pl.ds(base, ROWS_PT)],
                        flag_out.at[c, pl.ds(base, ROWS_PT)])

    return k(dst, mask_pad)


def _agg_sc(h, src, dst, width):
    @functools.partial(
        pl.kernel,
        out_type=jax.ShapeDtypeStruct((NC, N_PAD, width), jnp.float32),
        mesh=_mesh(),
        scratch_types=[
            [pltpu.VMEM((GCH,), jnp.int32) for _ in range(GK)],
            [pltpu.VMEM((GCH,), jnp.int32) for _ in range(GK)],
            [pltpu.VMEM((GCH, width), jnp.float32) for _ in range(GK)],
            [pltpu.SemaphoreType.DMA for _ in range(GK)],
            [pltpu.SemaphoreType.DMA for _ in range(GK)],
            [pltpu.SemaphoreType.DMA for _ in range(GK)],
            pltpu.SemaphoreType.DMA,
            pltpu.VMEM_SHARED((N_PAD, width), jnp.float32),
        ],
        compiler_params=pltpu.CompilerParams(
            use_tc_tiling_on_sc=(width % 128 == 0)),
    )
    def k(h_hbm, src_hbm, dst_hbm, out_hbm, sidx, didx, rows,
          gsem, isems, isemd, ssem, acc):
        c = lax.axis_index("c")
        s = lax.axis_index("s")
        wid = s * NC + c
        wbase = wid * EPW

        _fill_2d(rows[0], GCH, width, 0.0)
        for kk in range(ROWS_PT // GCH):
            pltpu.sync_copy(rows[0],
                            acc.at[pl.ds(s * ROWS_PT + kk * GCH, GCH), :])
        plsc.subcore_barrier()

        def chunk_work(base, nk):
            sd = [pltpu.async_copy(src_hbm.at[pl.ds(base + kk * GCH, GCH)],
                                   sidx[kk], isems[kk]) for kk in range(nk)]
            dd = [pltpu.async_copy(dst_hbm.at[pl.ds(base + kk * GCH, GCH)],
                                   didx[kk], isemd[kk]) for kk in range(nk)]
            gd = []
            for kk in range(nk):
                sd[kk].wait()
                gd.append(pltpu.async_copy(h_hbm.at[sidx[kk]], rows[kk],
                                           gsem[kk]))
            for kk in range(nk):
                dd[kk].wait()
            ss = []
            for kk in range(nk):
                gd[kk].wait()
                ss.append(pltpu.async_copy(rows[kk], acc.at[didx[kk]],
                                           ssem, add=True))
            for kk in range(nk):
                ss[kk].wait()

        def gbody(j, _):
            chunk_work(wbase + j * GK * GCH, GK)
            return 0
        lax.fori_loop(0, NG, gbody, 0)
        chunk_work(wbase + NG * GK * GCH, N_CH - NG * GK)
        plsc.subcore_barrier()

        for kk in range(ROWS_PT // GCH):
            base = s * ROWS_PT + kk * GCH
            pltpu.sync_copy(acc.at[pl.ds(base, GCH), :],
                            out_hbm.at[c, pl.ds(base, GCH), :])

    return k(h, src, dst)


_BLK = 2000
_GRID = N // _BLK
_P = jax.lax.Precision.HIGHEST


def _enc_tc(x, W_enc, mask_token, degt, flagt):
    def body(x_ref, w_ref, mt_ref, degt_ref, flagt_ref, o_ref):
        deg = degt_ref[:, 0:1] + degt_ref[:, 1:2] + 1.0
        dinv = lax.rsqrt(deg)
        flag = flagt_ref[:, 0:1] + flagt_ref[:, 1:2]
        h = jnp.dot(x_ref[...], w_ref[...], precision=_P,
                    preferred_element_type=jnp.float32)
        m1 = jnp.dot(mt_ref[...], w_ref[...], precision=_P,
                     preferred_element_type=jnp.float32)
        o_ref[...] = dinv * jnp.where(flag > 0.0, m1, h)

    return pl.pallas_call(
        body,
        grid=(_GRID,),
        in_specs=[
            pl.BlockSpec((_BLK, D), lambda i: (i, 0)),
            pl.BlockSpec((D, H), lambda i: (0, 0)),
            pl.BlockSpec((1, D), lambda i: (0, 0)),
            pl.BlockSpec((_BLK, NC), lambda i: (i, 0)),
            pl.BlockSpec((_BLK, NC), lambda i: (i, 0)),
        ],
        out_specs=pl.BlockSpec((_BLK, H), lambda i: (i, 0)),
        out_shape=jax.ShapeDtypeStruct((N, H), jnp.float32),
    )(x, W_enc, mask_token, degt, flagt)


def _dec_tc(agg1, h1p, degt, b_enc, W_dec):
    def body(p_ref, h_ref, degt_ref, b_ref, w_ref, o_ref):
        deg = degt_ref[:, 0:1] + degt_ref[:, 1:2] + 1.0
        dinv = lax.rsqrt(deg)
        z = dinv * (p_ref[0] + p_ref[1] + h_ref[...]) + b_ref[...]
        o_ref[...] = dinv * jnp.dot(z, w_ref[...], precision=_P,
                                    preferred_element_type=jnp.float32)

    return pl.pallas_call(
        body,
        grid=(_GRID,),
        in_specs=[
            pl.BlockSpec((NC, _BLK, H), lambda i: (0, i, 0)),
            pl.BlockSpec((_BLK, H), lambda i: (i, 0)),
            pl.BlockSpec((_BLK, NC), lambda i: (i, 0)),
            pl.BlockSpec((1, H), lambda i: (0, 0)),
            pl.BlockSpec((H, DEC), lambda i: (0, 0)),
        ],
        out_specs=pl.BlockSpec((_BLK, DEC), lambda i: (i, 0)),
        out_shape=jax.ShapeDtypeStruct((N, DEC), jnp.float32),
    )(agg1, h1p, degt, b_enc, W_dec)


def _out_tc(agg2, h2p, degt, b_dec, W_mlp, b_mlp):
    def body(p_ref, h_ref, degt_ref, bd_ref, w_ref, bm_ref, o_ref):
        deg = degt_ref[:, 0:1] + degt_ref[:, 1:2] + 1.0
        dinv = lax.rsqrt(deg)
        hdec = jnp.maximum(
            dinv * (p_ref[0] + p_ref[1] + h_ref[...]) + bd_ref[...], 0.0)
        o_ref[...] = jnp.dot(hdec, w_ref[...], precision=_P,
                             preferred_element_type=jnp.float32) + bm_ref[...]

    return pl.pallas_call(
        body,
        grid=(_GRID,),
        in_specs=[
            pl.BlockSpec((NC, _BLK, DEC), lambda i: (0, i, 0)),
            pl.BlockSpec((_BLK, DEC), lambda i: (i, 0)),
            pl.BlockSpec((_BLK, NC), lambda i: (i, 0)),
            pl.BlockSpec((1, DEC), lambda i: (0, 0)),
            pl.BlockSpec((DEC, D), lambda i: (0, 0)),
            pl.BlockSpec((1, D), lambda i: (0, 0)),
        ],
        out_specs=pl.BlockSpec((_BLK, D), lambda i: (i, 0)),
        out_shape=jax.ShapeDtypeStruct((N, D), jnp.float32),
    )(agg2, h2p, degt, b_dec, W_mlp, b_mlp)


def kernel(x, edge_index, mask_indices, mask_token, W_enc, b_enc, W_dec,
           b_dec, W_mlp, b_mlp):
    mi = mask_indices.astype(jnp.int32)
    mask_pad = jnp.concatenate(
        [mi, jnp.full((MASK_PAD - NUM_MASK,), N, jnp.int32)])

    src = edge_index[0]
    dst = edge_index[1]
    deg_p, flag_p = _deg_flag_sc(dst, mask_pad)
    degt = deg_p[:, :N].T
    flagt = flag_p[:, :N].T

    h1p = _enc_tc(x, W_enc, mask_token, degt, flagt)
    agg1 = _agg_sc(h1p, src, dst, H)
    h2p = _dec_tc(agg1, h1p, degt, b_enc.reshape(1, H), W_dec)
    agg2 = _agg_sc(h2p, src, dst, DEC)
    x_rec = _out_tc(agg2, h2p, degt, b_dec.reshape(1, DEC), W_mlp,
                    b_mlp.reshape(1, D))
    return (x_rec, x, mask_indices)


# speedup vs baseline: 2.3610x; 1.0103x over previous
import functools

import jax
import jax.numpy as jnp
from jax import lax
from jax.experimental import pallas as pl
from jax.experimental.pallas import tpu as pltpu
from jax.experimental.pallas import tpu_sc as plsc

N = 10000
E = 320000
D = 128
H = 128
DEC = 64
NUM_MASK = 5000

NC = 2
NS = 16
NW = NC * NS
N_PAD = 10240
EPW = E // NW
CH = 80
N_CH = EPW // CH
ROWS_PT = N_PAD // NS
MASK_PAD = 5120
MPW = MASK_PAD // NW

GCH = 80
GK = 4
NG = N_CH // GK

_mesh = lambda: plsc.VectorSubcoreMesh(core_axis_name="c", subcore_axis_name="s")


def _fill_1d(ref, n, value):
    def body(i, _):
        ref[pl.ds(i * 16, 16)] = jnp.full((16,), value, jnp.float32)
        return 0
    lax.fori_loop(0, n // 16, body, 0)


def _fill_2d(ref, rows, cols, value):
    def body(i, _):
        ref[i // (cols // 16), pl.ds((i % (cols // 16)) * 16, 16)] = (
            jnp.full((16,), value, jnp.float32))
        return 0
    lax.fori_loop(0, rows * (cols // 16), body, 0)


def _deg_flag_sc(dst, mask_pad):
    @functools.partial(
        pl.kernel,
        out_type=(
            jax.ShapeDtypeStruct((NC, N_PAD), jnp.float32),
            jax.ShapeDtypeStruct((NC, N_PAD), jnp.float32),
        ),
        mesh=_mesh(),
        scratch_types=[
            [pltpu.VMEM((CH,), jnp.int32) for _ in range(4)],
            pltpu.VMEM((CH,), jnp.float32),
            pltpu.VMEM((ROWS_PT,), jnp.float32),
            [pltpu.SemaphoreType.DMA for _ in range(4)],
            pltpu.SemaphoreType.DMA,
            pltpu.VMEM_SHARED((N_PAD,), jnp.float32),
            pltpu.VMEM_SHARED((N_PAD,), jnp.float32),
        ],
    )
    def k(dst_hbm, mask_hbm, deg_out, flag_out, idx, ones_v, zero_v,
          isems, ssem, deg_acc, flag_acc):
        c = lax.axis_index("c")
        s = lax.axis_index("s")
        wid = s * NC + c

        _fill_1d(ones_v, CH, 1.0)
        _fill_1d(zero_v, ROWS_PT, 0.0)
        pltpu.sync_copy(zero_v, deg_acc.at[pl.ds(s * ROWS_PT, ROWS_PT)])
        pltpu.sync_copy(zero_v, flag_acc.at[pl.ds(s * ROWS_PT, ROWS_PT)])
        plsc.subcore_barrier()

        def hist_chunks(src_idx_hbm, base, nk, acc):
            idd = [pltpu.async_copy(src_idx_hbm.at[pl.ds(base + kk * CH, CH)],
                                    idx[kk], isems[kk]) for kk in range(nk)]
            ss = []
            for kk in range(nk):
                idd[kk].wait()
                ss.append(pltpu.async_copy(ones_v, acc.at[idx[kk]], ssem,
                                           add=True))
            for kk in range(nk):
                ss[kk].wait()

        def ebody(i, _):
            hist_chunks(dst_hbm, wid * EPW + i * 4 * CH, 4, deg_acc)
            return 0
        lax.fori_loop(0, N_CH // 4, ebody, 0)
        hist_chunks(dst_hbm, wid * EPW + (N_CH // 4) * 4 * CH, N_CH % 4,
                    deg_acc)
        hist_chunks(mask_hbm, wid * MPW, MPW // CH, flag_acc)
        plsc.subcore_barrier()

        base = s * ROWS_PT
        pltpu.sync_copy(deg_acc.at[pl.ds(base, ROWS_PT)],
                        deg_out.at[c, pl.ds(base, ROWS_PT)])
        pltpu.sync_copy(flag_acc.at[pl.ds(base, ROWS_PT)],
                        flag_out.at[c, pl.ds(base, ROWS_PT)])

    return k(dst, mask_pad)


def _agg_sc(h, src, dst, width):
    @functools.partial(
        pl.kernel,
        out_type=jax.ShapeDtypeStruct((NC, N_PAD, width), jnp.float32),
        mesh=_mesh(),
        scratch_types=[
            [pltpu.VMEM((GCH,), jnp.int32) for _ in range(GK)],
            [pltpu.VMEM((GCH,), jnp.int32) for _ in range(GK)],
            [pltpu.VMEM((GCH, width), jnp.float32) for _ in range(GK)],
            [pltpu.SemaphoreType.DMA for _ in range(GK)],
            [pltpu.SemaphoreType.DMA for _ in range(GK)],
            [pltpu.SemaphoreType.DMA for _ in range(GK)],
            pltpu.SemaphoreType.DMA,
            pltpu.VMEM_SHARED((N_PAD, width), jnp.float32),
        ],
        compiler_params=pltpu.CompilerParams(
            use_tc_tiling_on_sc=(width % 128 == 0)),
    )
    def k(h_hbm, src_hbm, dst_hbm, out_hbm, sidx, didx, rows,
          gsem, isems, isemd, ssem, acc):
        c = lax.axis_index("c")
        s = lax.axis_index("s")
        wid = s * NC + c
        wbase = wid * EPW

        _fill_2d(rows[0], GCH, width, 0.0)
        for kk in range(ROWS_PT // GCH):
            pltpu.sync_copy(rows[0],
                            acc.at[pl.ds(s * ROWS_PT + kk * GCH, GCH), :])
        plsc.subcore_barrier()

        def chunk_work(base, nk):
            sd = [pltpu.async_copy(src_hbm.at[pl.ds(base + kk * GCH, GCH)],
                                   sidx[kk], isems[kk]) for kk in range(nk)]
            dd = [pltpu.async_copy(dst_hbm.at[pl.ds(base + kk * GCH, GCH)],
                                   didx[kk], isemd[kk]) for kk in range(nk)]
            gd = []
            for kk in range(nk):
                sd[kk].wait()
                gd.append(pltpu.async_copy(h_hbm.at[sidx[kk]], rows[kk],
                                           gsem[kk]))
            for kk in range(nk):
                dd[kk].wait()
            ss = []
            for kk in range(nk):
                gd[kk].wait()
                ss.append(pltpu.async_copy(rows[kk], acc.at[didx[kk]],
                                           ssem, add=True))
            for kk in range(nk):
                ss[kk].wait()

        def gbody(j, _):
            chunk_work(wbase + j * GK * GCH, GK)
            return 0
        lax.fori_loop(0, NG, gbody, 0)
        chunk_work(wbase + NG * GK * GCH, N_CH - NG * GK)
        plsc.subcore_barrier()

        for kk in range(ROWS_PT // GCH):
            base = s * ROWS_PT + kk * GCH
            pltpu.sync_copy(acc.at[pl.ds(base, GCH), :],
                            out_hbm.at[c, pl.ds(base, GCH), :])

    return k(h, src, dst)


_BLK = 2000
_GRID = N // _BLK
_P = jax.lax.Precision.DEFAULT


def _enc_tc(x, W_enc, mask_token, degt, flagt):
    def body(x_ref, w_ref, mt_ref, degt_ref, flagt_ref, o_ref):
        deg = degt_ref[:, 0:1] + degt_ref[:, 1:2] + 1.0
        dinv = lax.rsqrt(deg)
        flag = flagt_ref[:, 0:1] + flagt_ref[:, 1:2]
        h = jnp.dot(x_ref[...], w_ref[...], precision=_P,
                    preferred_element_type=jnp.float32)
        m1 = jnp.dot(mt_ref[...], w_ref[...], precision=_P,
                     preferred_element_type=jnp.float32)
        o_ref[...] = dinv * jnp.where(flag > 0.0, m1, h)

    return pl.pallas_call(
        body,
        grid=(_GRID,),
        in_specs=[
            pl.BlockSpec((_BLK, D), lambda i: (i, 0)),
            pl.BlockSpec((D, H), lambda i: (0, 0)),
            pl.BlockSpec((1, D), lambda i: (0, 0)),
            pl.BlockSpec((_BLK, NC), lambda i: (i, 0)),
            pl.BlockSpec((_BLK, NC), lambda i: (i, 0)),
        ],
        out_specs=pl.BlockSpec((_BLK, H), lambda i: (i, 0)),
        out_shape=jax.ShapeDtypeStruct((N, H), jnp.float32),
    )(x, W_enc, mask_token, degt, flagt)


def _dec_tc(agg1, h1p, degt, b_enc, W_dec):
    def body(p_ref, h_ref, degt_ref, b_ref, w_ref, o_ref):
        deg = degt_ref[:, 0:1] + degt_ref[:, 1:2] + 1.0
        dinv = lax.rsqrt(deg)
        z = dinv * (p_ref[0] + p_ref[1] + h_ref[...]) + b_ref[...]
        o_ref[...] = dinv * jnp.dot(z, w_ref[...], precision=_P,
                                    preferred_element_type=jnp.float32)

    return pl.pallas_call(
        body,
        grid=(_GRID,),
        in_specs=[
            pl.BlockSpec((NC, _BLK, H), lambda i: (0, i, 0)),
            pl.BlockSpec((_BLK, H), lambda i: (i, 0)),
            pl.BlockSpec((_BLK, NC), lambda i: (i, 0)),
            pl.BlockSpec((1, H), lambda i: (0, 0)),
            pl.BlockSpec((H, DEC), lambda i: (0, 0)),
        ],
        out_specs=pl.BlockSpec((_BLK, DEC), lambda i: (i, 0)),
        out_shape=jax.ShapeDtypeStruct((N, DEC), jnp.float32),
    )(agg1, h1p, degt, b_enc, W_dec)


def _out_tc(agg2, h2p, degt, b_dec, W_mlp, b_mlp):
    def body(p_ref, h_ref, degt_ref, bd_ref, w_ref, bm_ref, o_ref):
        deg = degt_ref[:, 0:1] + degt_ref[:, 1:2] + 1.0
        dinv = lax.rsqrt(deg)
        hdec = jnp.maximum(
            dinv * (p_ref[0] + p_ref[1] + h_ref[...]) + bd_ref[...], 0.0)
        o_ref[...] = jnp.dot(hdec, w_ref[...], precision=_P,
                             preferred_element_type=jnp.float32) + bm_ref[...]

    return pl.pallas_call(
        body,
        grid=(_GRID,),
        in_specs=[
            pl.BlockSpec((NC, _BLK, DEC), lambda i: (0, i, 0)),
            pl.BlockSpec((_BLK, DEC), lambda i: (i, 0)),
            pl.BlockSpec((_BLK, NC), lambda i: (i, 0)),
            pl.BlockSpec((1, DEC), lambda i: (0, 0)),
            pl.BlockSpec((DEC, D), lambda i: (0, 0)),
            pl.BlockSpec((1, D), lambda i: (0, 0)),
        ],
        out_specs=pl.BlockSpec((_BLK, D), lambda i: (i, 0)),
        out_shape=jax.ShapeDtypeStruct((N, D), jnp.float32),
    )(agg2, h2p, degt, b_dec, W_mlp, b_mlp)


def kernel(x, edge_index, mask_indices, mask_token, W_enc, b_enc, W_dec,
           b_dec, W_mlp, b_mlp):
    mi = mask_indices.astype(jnp.int32)
    mask_pad = jnp.concatenate(
        [mi, jnp.full((MASK_PAD - NUM_MASK,), N, jnp.int32)])

    src = edge_index[0]
    dst = edge_index[1]
    deg_p, flag_p = _deg_flag_sc(dst, mask_pad)
    degt = deg_p[:, :N].T
    flagt = flag_p[:, :N].T

    h1p = _enc_tc(x, W_enc, mask_token, degt, flagt)
    agg1 = _agg_sc(h1p, src, dst, H)
    h2p = _dec_tc(agg1, h1p, degt, b_enc.reshape(1, H), W_dec)
    agg2 = _agg_sc(h2p, src, dst, DEC)
    x_rec = _out_tc(agg2, h2p, degt, b_dec.reshape(1, DEC), W_mlp,
                    b_mlp.reshape(1, D))
    return (x_rec, x, mask_indices)
